# baseline (speedup 1.0000x reference)
import jax
import jax.numpy as jnp
from jax import lax
from jax.experimental import pallas as pl
from jax.experimental.pallas import tpu as pltpu
from jax.experimental.pallas import tpu_sc as plsc

N = 10000
E = 320000
D = 128
OUT = 128

NC = 2
NS = 16
NW = NC * NS
NP = 10240
STRIPE = NP // NS
ZB = 128
EW = 10240
EPAD = EW * NW

CH = 64
NPHASE = 8
NBUF = 4
EW0 = 16384
EW1 = 4096
HCH0 = EW0 // CH // NPHASE
HCH1 = EW1 // CH // NPHASE
GR0 = HCH0 // NBUF
GR1 = HCH1 // NBUF
FC1 = 1
FC2 = 0

DCH = 128
DNPHASE = 4
DHCHUNK = (EW // DCH) // DNPHASE

BLK = 640
OBLK = 2000


def _sc_propagate(fastc):
    mesh = plsc.VectorSubcoreMesh(core_axis_name="c", subcore_axis_name="s")

    out_type = [jax.ShapeDtypeStruct((NC, NP, D), jnp.float32)]
    scratch = [
        pltpu.VMEM((HCH0, CH), jnp.int32),
        pltpu.VMEM((HCH0, CH), jnp.int32),
        [pltpu.VMEM((CH, D), jnp.float32) for _ in range(NBUF)],
        [pltpu.VMEM((CH,), jnp.int32) for _ in range(NBUF)],
        [pltpu.SemaphoreType.DMA for _ in range(NBUF)],
        [pltpu.SemaphoreType.DMA for _ in range(NBUF)],
        pltpu.VMEM_SHARED((NP, D), jnp.float32),
    ]

    def body(table_h, src_h, dst_h, z64_h, out_h, src_v, dst_v, rows, dchs,
             gsems, ssems, acc):
        c = lax.axis_index("c")
        s = lax.axis_index("s")
        wid = s * NC + c
        base = s * STRIPE
        is_fast = c == fastc
        hch = jnp.where(is_fast, HCH0, HCH1)
        groups = jnp.where(is_fast, GR0, GR1)
        pltpu.sync_copy(z64_h, rows[0])
        for j in range(STRIPE // CH):
            pltpu.sync_copy(rows[0], acc.at[pl.ds(base + j * CH, CH)])
        plsc.subcore_barrier()

        def stage(jj, b):
            for k in range(CH // 16):
                dchs[b][pl.ds(k * 16, 16)] = dst_v[jj, pl.ds(k * 16, 16)]

        def gather(jj, b):
            pltpu.async_copy(table_h.at[src_v.at[jj]], rows[b], gsems[b])

        def wait_gather(jj, b):
            pltpu.make_async_copy(table_h.at[src_v.at[jj]], rows[b], gsems[b]).wait()

        def scatter(b):
            pltpu.async_copy(rows[b], acc.at[dchs[b]], ssems[b], add=True)

        def wait_scatter(b):
            pltpu.make_async_copy(rows[b], acc.at[dchs[b]], ssems[b]).wait()

        first = True
        for ph in range(NPHASE):
            pltpu.sync_copy(src_h.at[wid * NPHASE + ph], src_v)
            pltpu.sync_copy(dst_h.at[wid * NPHASE + ph], dst_v)

            for u in range(NBUF - 1):
                if not first:
                    wait_scatter(u)
                stage(u, u)
                gather(u, u)

            guard_first = first

            def group(t, carry):
                for u in range(NBUF):
                    j = t * NBUF + u
                    wait_gather(j, u)
                    scatter(u)
                    jn = j + NBUF - 1
                    bn = (u + NBUF - 1) % NBUF

                    @pl.when(jn < hch)
                    def _(u=u, jn=jn, bn=bn):
                        if guard_first and u == 0:
                            @pl.when(t > 0)
                            def _w():
                                wait_scatter(bn)
                        else:
                            wait_scatter(bn)
                        stage(jn, bn)
                        gather(jn, bn)
                return carry

            lax.fori_loop(0, groups, group, 0)
            first = False

        for u in range(NBUF):
            wait_scatter(u)
        plsc.subcore_barrier()

        for j in range(STRIPE // CH):
            r = base + j * CH
            pltpu.sync_copy(acc.at[pl.ds(r, CH)], rows[0])
            pltpu.sync_copy(rows[0], out_h.at[c, pl.ds(r, CH)])

    return pl.kernel(body, out_type=out_type, mesh=mesh, scratch_types=scratch)


def _sc_degree():
    mesh = plsc.VectorSubcoreMesh(core_axis_name="c", subcore_axis_name="s")

    out_type = [jax.ShapeDtypeStruct((NC, NP, D), jnp.float32)]
    scratch = [
        pltpu.VMEM((DHCHUNK, DCH), jnp.int32),
        pltpu.VMEM((ZB, D), jnp.float32),
        pltpu.VMEM((DCH,), jnp.int32),
        pltpu.VMEM_SHARED((NP, D), jnp.float32),
    ]

    def body(dst_h, z128_h, one128_h, deg_h, dst_v, rows_v, dchunk, dacc):
        c = lax.axis_index("c")
        s = lax.axis_index("s")
        wid = s * NC + c
        base = s * STRIPE

        pltpu.sync_copy(z128_h, rows_v)
        for j in range(STRIPE // ZB):
            pltpu.sync_copy(rows_v, dacc.at[pl.ds(base + j * ZB, ZB)])
        pltpu.sync_copy(one128_h, rows_v)
        plsc.subcore_barrier()

        def step(j, carry):
            for k in range(DCH // 16):
                dchunk[pl.ds(k * 16, 16)] = dst_v[j, pl.ds(k * 16, 16)]
            pltpu.sync_copy(rows_v, dacc.at[dchunk], add=True)
            return carry

        for ph in range(DNPHASE):
            pltpu.sync_copy(dst_h.at[wid * DNPHASE + ph], dst_v)
            lax.fori_loop(0, DHCHUNK, step, 0)

        plsc.subcore_barrier()

        for j in range(STRIPE // ZB):
            r = base + j * ZB
            pltpu.sync_copy(dacc.at[pl.ds(r, ZB)], rows_v)
            pltpu.sync_copy(rows_v, deg_h.at[c, pl.ds(r, ZB)])

    return pl.kernel(body, out_type=out_type, mesh=mesh, scratch_types=scratch)


def _tc_h1_body(p0, p1, d0, d1, h1):
    cnt = d0[...][0][:, 0:1] + d1[...][0][:, 0:1]
    inv = 1.0 / jnp.maximum(cnt, 1.0)
    h1[...] = (p0[...][0] + p1[...][0]) * inv


def _tc_out_body(p0, p1, d0, d1, x, h1, w0, w1, w2, b, out):
    cnt = d0[...][0][:, 0:1] + d1[...][0][:, 0:1]
    inv = 1.0 / jnp.maximum(cnt, 1.0)
    h2 = (p0[...][0] + p1[...][0]) * inv
    acc = jnp.dot(x[...], w0[...], precision=lax.Precision.HIGHEST)
    acc += jnp.dot(h1[...], w1[...], precision=lax.Precision.HIGHEST)
    acc += jnp.dot(h2, w2[...], precision=lax.Precision.HIGHEST)
    out[...] = acc + b[...]


def _row_spec(rows, width):
    return pl.BlockSpec((rows, width), lambda i: (i, 0))


def _core_spec(core, rows, width):
    return pl.BlockSpec((1, rows, width), lambda i, core=core: (core, i, 0))


def _const_spec(shape):
    return pl.BlockSpec(shape, lambda i: (0, 0))


def kernel(x, edge_index, W, b):
    src = edge_index[0]
    dst = edge_index[1]
    pad = EPAD - E
    srcf = jnp.concatenate([src, jnp.zeros((pad,), jnp.int32)])
    dstf = jnp.concatenate([dst, jnp.full((pad,), N, jnp.int32)])

    def split(flat, dummy, fastc):
        nfast = 16 * EW0
        fastb = flat[:nfast].reshape(16, NPHASE, HCH0, CH)
        slowb = flat[nfast:].reshape(16, NPHASE, HCH1, CH)
        arr = jnp.full((NW, NPHASE, HCH0, CH), dummy, jnp.int32)
        arr = arr.at[fastc::NC].set(fastb)
        arr = arr.at[1 - fastc::NC, :, :HCH1].set(slowb)
        return arr.reshape(NW * NPHASE, HCH0, CH)

    srcp1 = split(srcf, 0, FC1)
    dstp1 = split(dstf, N, FC1)
    srcp2 = split(srcf, 0, FC2)
    dstp2 = split(dstf, N, FC2)
    dstp_deg = dstf.reshape(NW * DNPHASE, DHCHUNK, DCH)

    z64 = jnp.zeros((CH, D), jnp.float32)
    z128 = jnp.zeros((ZB, D), jnp.float32)
    one128 = jnp.ones((ZB, D), jnp.float32)

    (parts1,) = _sc_propagate(FC1)(x, srcp1, dstp1, z64)
    (degs,) = _sc_degree()(dstp_deg, z128, one128)

    h1 = pl.pallas_call(
        _tc_h1_body,
        grid=(NP // BLK,),
        in_specs=[
            _core_spec(0, BLK, D), _core_spec(1, BLK, D),
            _core_spec(0, BLK, D), _core_spec(1, BLK, D),
        ],
        out_specs=_row_spec(BLK, D),
        out_shape=jax.ShapeDtypeStruct((NP, D), jnp.float32),
    )(parts1, parts1, degs, degs)

    (parts2,) = _sc_propagate(FC2)(h1, srcp2, dstp2, z64)

    w0 = W[:, :D].T
    w1 = W[:, D:2 * D].T
    w2 = W[:, 2 * D:].T
    b2 = b.reshape(1, OUT)

    out = pl.pallas_call(
        _tc_out_body,
        grid=(N // OBLK,),
        in_specs=[
            _core_spec(0, OBLK, D), _core_spec(1, OBLK, D),
            _core_spec(0, OBLK, D), _core_spec(1, OBLK, D),
            _row_spec(OBLK, D), _row_spec(OBLK, D),
            _const_spec((D, OUT)), _const_spec((D, OUT)), _const_spec((D, OUT)),
            _const_spec((1, OUT)),
        ],
        out_specs=_row_spec(OBLK, OUT),
        out_shape=jax.ShapeDtypeStruct((N, OUT), jnp.float32),
    )(parts2, parts2, degs, degs, x, h1, w0, w1, w2, b2)

    return out


# speedup vs baseline: 1.1360x; 1.1360x over previous
import jax
import jax.numpy as jnp
from jax import lax
from jax.experimental import pallas as pl
from jax.experimental.pallas import tpu as pltpu
from jax.experimental.pallas import tpu_sc as plsc

N = 10000
E = 320000
D = 128
OUT = 128

NC = 2
NS = 16
NW = NC * NS
NP = 10240
STRIPE = NP // NS
ZB = 128
EW = 10240
EPAD = EW * NW

CH = 64
NPHASE = 8
NBUF = 5
NCHUNK = EW // CH
HCHUNK = NCHUNK // NPHASE
GROUPS = HCHUNK // NBUF

DCH = 128
DNPHASE = 4
DHCHUNK = (EW // DCH) // DNPHASE

BLK = 640
OBLK = 2000


def _sc_propagate():
    mesh = plsc.VectorSubcoreMesh(core_axis_name="c", subcore_axis_name="s")

    out_type = [jax.ShapeDtypeStruct((NC, NP, D), jnp.float32)]
    scratch = [
        pltpu.VMEM((HCHUNK, CH), jnp.int32),
        pltpu.VMEM((HCHUNK, CH), jnp.int32),
        [pltpu.VMEM((CH, D), jnp.float32) for _ in range(NBUF)],
        [pltpu.VMEM((CH,), jnp.int32) for _ in range(NBUF)],
        [pltpu.SemaphoreType.DMA for _ in range(NBUF)],
        [pltpu.SemaphoreType.DMA for _ in range(NBUF)],
        pltpu.VMEM_SHARED((NP, D), jnp.float32),
    ]

    def body(table_h, src_h, dst_h, z64_h, out_h, src_v, dst_v, rows, dchs,
             gsems, ssems, acc):
        c = lax.axis_index("c")
        s = lax.axis_index("s")
        wid = s * NC + c
        base = s * STRIPE
        pltpu.sync_copy(z64_h, rows[0])
        for j in range(STRIPE // CH):
            pltpu.sync_copy(rows[0], acc.at[pl.ds(base + j * CH, CH)])
        plsc.subcore_barrier()

        def stage(jj, b):
            for k in range(CH // 16):
                dchs[b][pl.ds(k * 16, 16)] = dst_v[jj, pl.ds(k * 16, 16)]

        def gather(jj, b):
            pltpu.async_copy(table_h.at[src_v.at[jj]], rows[b], gsems[b])

        def wait_gather(jj, b):
            pltpu.make_async_copy(table_h.at[src_v.at[jj]], rows[b], gsems[b]).wait()

        def scatter(b):
            pltpu.async_copy(rows[b], acc.at[dchs[b]], ssems[b], add=True)

        def wait_scatter(b):
            pltpu.make_async_copy(rows[b], acc.at[dchs[b]], ssems[b]).wait()

        first = True
        for ph in range(NPHASE):
            pltpu.sync_copy(src_h.at[wid * NPHASE + ph], src_v)
            pltpu.sync_copy(dst_h.at[wid * NPHASE + ph], dst_v)

            for u in range(NBUF - 1):
                if not first:
                    wait_scatter(u)
                stage(u, u)
                gather(u, u)

            guard_first = first

            def group(t, carry):
                for u in range(NBUF):
                    j = t * NBUF + u
                    wait_gather(j, u)
                    scatter(u)
                    jn = j + NBUF - 1
                    bn = (u + NBUF - 1) % NBUF

                    @pl.when(jn < HCHUNK)
                    def _(u=u, jn=jn, bn=bn):
                        if guard_first and u == 0:
                            @pl.when(t > 0)
                            def _w():
                                wait_scatter(bn)
                        else:
                            wait_scatter(bn)
                        stage(jn, bn)
                        gather(jn, bn)
                return carry

            lax.fori_loop(0, GROUPS, group, 0)
            first = False

        for u in range(NBUF):
            wait_scatter(u)
        plsc.subcore_barrier()

        for j in range(STRIPE // CH):
            r = base + j * CH
            pltpu.sync_copy(acc.at[pl.ds(r, CH)], rows[0])
            pltpu.sync_copy(rows[0], out_h.at[c, pl.ds(r, CH)])

    return pl.kernel(body, out_type=out_type, mesh=mesh, scratch_types=scratch)


def _sc_degree():
    mesh = plsc.VectorSubcoreMesh(core_axis_name="c", subcore_axis_name="s")

    out_type = [jax.ShapeDtypeStruct((NC, NP, D), jnp.float32)]
    scratch = [
        pltpu.VMEM((DHCHUNK, DCH), jnp.int32),
        pltpu.VMEM((ZB, D), jnp.float32),
        pltpu.VMEM((DCH,), jnp.int32),
        pltpu.VMEM_SHARED((NP, D), jnp.float32),
    ]

    def body(dst_h, z128_h, one128_h, deg_h, dst_v, rows_v, dchunk, dacc):
        c = lax.axis_index("c")
        s = lax.axis_index("s")
        wid = s * NC + c
        base = s * STRIPE

        pltpu.sync_copy(z128_h, rows_v)
        for j in range(STRIPE // ZB):
            pltpu.sync_copy(rows_v, dacc.at[pl.ds(base + j * ZB, ZB)])
        pltpu.sync_copy(one128_h, rows_v)
        plsc.subcore_barrier()

        def step(j, carry):
            for k in range(DCH // 16):
                dchunk[pl.ds(k * 16, 16)] = dst_v[j, pl.ds(k * 16, 16)]
            pltpu.sync_copy(rows_v, dacc.at[dchunk], add=True)
            return carry

        for ph in range(DNPHASE):
            pltpu.sync_copy(dst_h.at[wid * DNPHASE + ph], dst_v)
            lax.fori_loop(0, DHCHUNK, step, 0)

        plsc.subcore_barrier()

        for j in range(STRIPE // ZB):
            r = base + j * ZB
            pltpu.sync_copy(dacc.at[pl.ds(r, ZB)], rows_v)
            pltpu.sync_copy(rows_v, deg_h.at[c, pl.ds(r, ZB)])

    return pl.kernel(body, out_type=out_type, mesh=mesh, scratch_types=scratch)


def _tc_h1_body(p0, p1, d0, d1, h1):
    cnt = d0[...][0][:, 0:1] + d1[...][0][:, 0:1]
    inv = 1.0 / jnp.maximum(cnt, 1.0)
    h1[...] = (p0[...][0] + p1[...][0]) * inv


def _tc_out_body(p0, p1, d0, d1, x, h1, w0, w1, w2, b, out):
    cnt = d0[...][0][:, 0:1] + d1[...][0][:, 0:1]
    inv = 1.0 / jnp.maximum(cnt, 1.0)
    h2 = (p0[...][0] + p1[...][0]) * inv
    acc = jnp.dot(x[...], w0[...], precision=lax.Precision.HIGHEST)
    acc += jnp.dot(h1[...], w1[...], precision=lax.Precision.HIGHEST)
    acc += jnp.dot(h2, w2[...], precision=lax.Precision.HIGHEST)
    out[...] = acc + b[...]


def _row_spec(rows, width):
    return pl.BlockSpec((rows, width), lambda i: (i, 0))


def _core_spec(core, rows, width):
    return pl.BlockSpec((1, rows, width), lambda i, core=core: (core, i, 0))


def _const_spec(shape):
    return pl.BlockSpec(shape, lambda i: (0, 0))


def kernel(x, edge_index, W, b):
    src = edge_index[0]
    dst = edge_index[1]
    pad = EPAD - E
    srcf = jnp.concatenate([src, jnp.zeros((pad,), jnp.int32)])
    dstf = jnp.concatenate([dst, jnp.full((pad,), N, jnp.int32)])

    srcp = srcf.reshape(NW * NPHASE, HCHUNK, CH)
    dstp = dstf.reshape(NW * NPHASE, HCHUNK, CH)
    dstp_deg = dstf.reshape(NW * DNPHASE, DHCHUNK, DCH)

    z64 = jnp.zeros((CH, D), jnp.float32)
    z128 = jnp.zeros((ZB, D), jnp.float32)
    one128 = jnp.ones((ZB, D), jnp.float32)

    (parts1,) = _sc_propagate()(x, srcp, dstp, z64)
    (degs,) = _sc_degree()(dstp_deg, z128, one128)

    h1 = pl.pallas_call(
        _tc_h1_body,
        grid=(NP // BLK,),
        in_specs=[
            _core_spec(0, BLK, D), _core_spec(1, BLK, D),
            _core_spec(0, BLK, D), _core_spec(1, BLK, D),
        ],
        out_specs=_row_spec(BLK, D),
        out_shape=jax.ShapeDtypeStruct((NP, D), jnp.float32),
    )(parts1, parts1, degs, degs)

    (parts2,) = _sc_propagate()(h1, srcp, dstp, z64)

    w0 = W[:, :D].T
    w1 = W[:, D:2 * D].T
    w2 = W[:, 2 * D:].T
    b2 = b.reshape(1, OUT)

    out = pl.pallas_call(
        _tc_out_body,
        grid=(N // OBLK,),
        in_specs=[
            _core_spec(0, OBLK, D), _core_spec(1, OBLK, D),
            _core_spec(0, OBLK, D), _core_spec(1, OBLK, D),
            _row_spec(OBLK, D), _row_spec(OBLK, D),
            _const_spec((D, OUT)), _const_spec((D, OUT)), _const_spec((D, OUT)),
            _const_spec((1, OUT)),
        ],
        out_specs=_row_spec(OBLK, OUT),
        out_shape=jax.ShapeDtypeStruct((N, OUT), jnp.float32),
    )(parts2, parts2, degs, degs, x, h1, w0, w1, w2, b2)

    return out
